# baseline (speedup 1.0000x reference)
import functools

import jax
import jax.numpy as jnp
from jax import lax
from jax.experimental import pallas as pl
from jax.experimental.pallas import tpu as pltpu
from jax.experimental.pallas import tpu_sc as plsc

N = 10000
E = 320000
SC_CORES = 2
SC_SUBCORES = 16
LANES = 16
NW = SC_CORES * SC_SUBCORES
BLK = 96
BPW = 105
EPW = BPW * BLK
E_PAD = NW * EPW
PADN = E_PAD - E
NBUF = 2
NP = 10112
RPT = NP // SC_SUBCORES
ROW_BLK = 632
GRID = NP // ROW_BLK

_MESH = plsc.VectorSubcoreMesh(core_axis_name="c", subcore_axis_name="s",
                               num_cores=SC_CORES, num_subcores=SC_SUBCORES)


def _deg_body(dst_hbm, out_hbm, dst_v, hist_v):
    c = lax.axis_index("c")
    s = lax.axis_index("s")
    wid = c * SC_SUBCORES + s
    pltpu.sync_copy(dst_hbm.at[wid], dst_v)
    zero16 = jnp.zeros((LANES,), jnp.float32)
    ones16 = jnp.ones((LANES,), jnp.float32)

    def zb(i, carry):
        hist_v[pl.ds(i * LANES, LANES)] = zero16
        return carry

    lax.fori_loop(0, NP // LANES, zb, 0)

    def hb(i, carry):
        idx = dst_v[pl.ds(i * LANES, LANES)]
        plsc.addupdate_scatter(hist_v, [idx], ones16)
        return carry

    lax.fori_loop(0, EPW // LANES, hb, 0)
    pltpu.sync_copy(hist_v, out_hbm.at[wid])


_SC_PARAMS = pltpu.CompilerParams(needs_layout_passes=False)

_sc_deg = pl.kernel(
    _deg_body,
    out_type=jax.ShapeDtypeStruct((NW, NP), jnp.float32),
    mesh=_MESH,
    compiler_params=_SC_PARAMS,
    scratch_types=[
        pltpu.VMEM((EPW,), jnp.int32),
        pltpu.VMEM((NP,), jnp.float32),
    ],
)


def _make_spmm(nc):
    def body(src_hbm, dst_hbm, zeros_hbm, *rest):
        xs = rest[:nc]
        out_hbm = rest[nc]
        src_v, dst_v, gb0, gb1, acc = rest[nc + 1:nc + 6]
        gsem0, gsem1 = rest[nc + 6:nc + 8]
        gb = (gb0, gb1)
        gsem = (gsem0, gsem1)
        c = lax.axis_index("c")
        s = lax.axis_index("s")
        wid = c * SC_SUBCORES + s
        pltpu.async_copy(src_hbm.at[wid], src_v, gsem0)
        pltpu.async_copy(dst_hbm.at[wid], dst_v, gsem1)
        idx_pending = True

        for j in range(nc):
            xs_j = xs[j]
            pltpu.sync_copy(zeros_hbm, acc.at[pl.ds(s * RPT, RPT), :])
            if idx_pending:
                pltpu.make_async_copy(src_hbm.at[wid], src_v, gsem0).wait()
                pltpu.make_async_copy(dst_hbm.at[wid], dst_v, gsem1).wait()
                idx_pending = False
            plsc.subcore_barrier()

            for k in range(NBUF):
                pltpu.async_copy(xs_j.at[src_v.at[pl.ds(k * BLK, BLK)]],
                                 gb[k], gsem[k])

            def rnd(r, carry):
                for k in range(NBUF):
                    b = r * NBUF + k
                    pltpu.make_async_copy(
                        xs_j.at[src_v.at[pl.ds(b * BLK, BLK)]], gb[k],
                        gsem[k]).wait()
                    pltpu.sync_copy(gb[k], acc.at[dst_v.at[b]], add=True)
                    pltpu.async_copy(
                        xs_j.at[src_v.at[pl.ds((b + NBUF) * BLK, BLK)]],
                        gb[k], gsem[k])
                return carry

            lax.fori_loop(0, BPW // NBUF - 1, rnd, 0)
            for b in range((BPW // NBUF - 1) * NBUF, BPW):
                k = b % NBUF
                pltpu.make_async_copy(
                    xs_j.at[src_v.at[pl.ds(b * BLK, BLK)]], gb[k],
                    gsem[k]).wait()
                pltpu.sync_copy(gb[k], acc.at[dst_v.at[b]], add=True)
                if b + NBUF < BPW:
                    pltpu.async_copy(
                        xs_j.at[src_v.at[pl.ds((b + NBUF) * BLK, BLK)]],
                        gb[k], gsem[k])
            plsc.subcore_barrier()
            pltpu.sync_copy(acc.at[pl.ds(s * RPT, RPT), :],
                            out_hbm.at[c, j, pl.ds(s * RPT, RPT), :])

    return pl.kernel(
        body,
        out_type=jax.ShapeDtypeStruct((SC_CORES, nc, NP, 128), jnp.float32),
        mesh=_MESH,
        compiler_params=_SC_PARAMS,
        scratch_types=(
            [pltpu.VMEM((EPW,), jnp.int32),
             pltpu.VMEM((BPW, BLK), jnp.int32)]
            + [pltpu.VMEM((BLK, 128), jnp.float32)] * 2
            + [pltpu.VMEM_SHARED((NP, 128), jnp.float32)]
            + [pltpu.SemaphoreType.DMA] * 2
        ),
    )


_sc_spmm1 = _make_spmm(1)


def _gelu(x):
    return 0.5 * x * (1.0 + lax.erf(x * 0.7071067811865476))


def _id_mask():
    rid = (lax.broadcasted_iota(jnp.int32, (ROW_BLK, 1), 0)
           + pl.program_id(0) * ROW_BLK)
    return ((rid >= PADN) & (rid < N)).astype(jnp.float32)


def _tc_a_body(degp_ref, x_ref, dinv_ref, x1s_ref):
    deg = jnp.sum(degp_ref[0], axis=1) + _id_mask()[:, 0]
    dinv = jnp.where(deg > 0, lax.rsqrt(jnp.maximum(deg, 1e-12)), 0.0)
    dinv_ref[...] = dinv[:, None]
    x1s_ref[...] = x_ref[...] * dinv[:, None]


def _tc_a(degp, x_pad):
    return pl.pallas_call(
        _tc_a_body,
        grid=(GRID,),
        in_specs=[
            pl.BlockSpec((1, ROW_BLK, NW), lambda i: (i, 0, 0)),
            pl.BlockSpec((ROW_BLK, 128), lambda i: (i, 0)),
        ],
        out_specs=[
            pl.BlockSpec((ROW_BLK, 1), lambda i: (i, 0)),
            pl.BlockSpec((ROW_BLK, 128), lambda i: (i, 0)),
        ],
        out_shape=[
            jax.ShapeDtypeStruct((NP, 1), jnp.float32),
            jax.ShapeDtypeStruct((NP, 128), jnp.float32),
        ],
    )(degp, x_pad)


def _sum_partials(p_ref, nc, x_refs):
    m = _id_mask()
    g = p_ref[0] + p_ref[1]
    return jnp.concatenate(
        [g[j] + m * x_refs[j][...] for j in range(nc)], axis=1)


def _tc_b_body(p_ref, x0_ref, dinv_ref, w1_ref, b1_ref, *out_refs):
    g = _sum_partials(p_ref, 1, [x0_ref])
    dinv = dinv_ref[...]
    h = jnp.dot(g * dinv, w1_ref[...],
                preferred_element_type=jnp.float32) + b1_ref[...]
    h = _gelu(h) * dinv
    for j in range(4):
        out_refs[j][...] = h[:, j * 128:(j + 1) * 128]


def _tc_b(p1, x1s, dinv, W1, b1):
    return pl.pallas_call(
        _tc_b_body,
        grid=(GRID,),
        in_specs=[
            pl.BlockSpec((SC_CORES, 1, ROW_BLK, 128), lambda i: (0, 0, i, 0)),
            pl.BlockSpec((ROW_BLK, 128), lambda i: (i, 0)),
            pl.BlockSpec((ROW_BLK, 1), lambda i: (i, 0)),
            pl.BlockSpec((128, 512), lambda i: (0, 0)),
            pl.BlockSpec((512,), lambda i: (0,)),
        ],
        out_specs=[pl.BlockSpec((ROW_BLK, 128), lambda i: (i, 0))] * 4,
        out_shape=[jax.ShapeDtypeStruct((NP, 128), jnp.float32)] * 4,
    )(p1, x1s, dinv, W1, b1)


def _tc_l2_init_body(p_ref, x_ref, dinv_ref, w_ref, b2_ref, out_ref):
    g = _sum_partials(p_ref, 1, [x_ref]) * dinv_ref[...]
    out_ref[...] = jnp.dot(g, w_ref[...],
                           preferred_element_type=jnp.float32) + b2_ref[...]


def _tc_l2_step_body(acc_ref, p_ref, x_ref, dinv_ref, w_ref, out_ref):
    g = _sum_partials(p_ref, 1, [x_ref]) * dinv_ref[...]
    out_ref[...] = acc_ref[...] + jnp.dot(
        g, w_ref[...], preferred_element_type=jnp.float32)


_P_SPEC = pl.BlockSpec((SC_CORES, 1, ROW_BLK, 128), lambda i: (0, 0, i, 0))
_X_SPEC = pl.BlockSpec((ROW_BLK, 128), lambda i: (i, 0))
_D_SPEC = pl.BlockSpec((ROW_BLK, 1), lambda i: (i, 0))
_A2_SPEC = pl.BlockSpec((ROW_BLK, 768), lambda i: (i, 0))


def _tc_l2_init(p, xj, dinv, Wj, b2):
    return pl.pallas_call(
        _tc_l2_init_body,
        grid=(GRID,),
        in_specs=[_P_SPEC, _X_SPEC, _D_SPEC,
                  pl.BlockSpec((128, 768), lambda i: (0, 0)),
                  pl.BlockSpec((768,), lambda i: (0,))],
        out_specs=_A2_SPEC,
        out_shape=jax.ShapeDtypeStruct((NP, 768), jnp.float32),
    )(p, xj, dinv, Wj, b2)


def _tc_l2_step(acc, p, xj, dinv, Wj):
    return pl.pallas_call(
        _tc_l2_step_body,
        grid=(GRID,),
        in_specs=[_A2_SPEC, _P_SPEC, _X_SPEC, _D_SPEC,
                  pl.BlockSpec((128, 768), lambda i: (0, 0))],
        out_specs=_A2_SPEC,
        out_shape=jax.ShapeDtypeStruct((NP, 768), jnp.float32),
    )(acc, p, xj, dinv, Wj)


def _tc_c2_body(acc_ref, dinv_ref, w3_ref, *out_refs):
    h = _gelu(acc_ref[...])
    t = jnp.dot(h, w3_ref[...], preferred_element_type=jnp.float32)
    t = t * dinv_ref[...]
    for j in range(4):
        out_refs[j][...] = t[:, j * 128:(j + 1) * 128]


def _tc_c2(acc2, dinv, W3):
    return pl.pallas_call(
        _tc_c2_body,
        grid=(GRID,),
        in_specs=[_A2_SPEC, _D_SPEC,
                  pl.BlockSpec((768, 512), lambda i: (0, 0))],
        out_specs=[_X_SPEC] * 4,
        out_shape=[jax.ShapeDtypeStruct((NP, 128), jnp.float32)] * 4,
    )(acc2, dinv, W3)


def _tc_l3_init_body(p_ref, x_ref, dinv_ref, b3_ref, w_ref, out_ref):
    g = _sum_partials(p_ref, 1, [x_ref]) * dinv_ref[...]
    h = _gelu(g + b3_ref[...])
    out_ref[...] = jnp.dot(h, w_ref[...], preferred_element_type=jnp.float32)


def _tc_l3_step_body(acc_ref, p_ref, x_ref, dinv_ref, b3_ref, w_ref, out_ref):
    g = _sum_partials(p_ref, 1, [x_ref]) * dinv_ref[...]
    h = _gelu(g + b3_ref[...])
    out_ref[...] = acc_ref[...] + jnp.dot(
        h, w_ref[...], preferred_element_type=jnp.float32)


def _tc_l3_init(p, xj, dinv, b3j, W4j):
    return pl.pallas_call(
        _tc_l3_init_body,
        grid=(GRID,),
        in_specs=[_P_SPEC, _X_SPEC, _D_SPEC,
                  pl.BlockSpec((128,), lambda i: (0,)),
                  pl.BlockSpec((128, 128), lambda i: (0, 0))],
        out_specs=_X_SPEC,
        out_shape=jax.ShapeDtypeStruct((NP, 128), jnp.float32),
    )(p, xj, dinv, b3j, W4j)


def _tc_l3_step(acc, p, xj, dinv, b3j, W4j):
    return pl.pallas_call(
        _tc_l3_step_body,
        grid=(GRID,),
        in_specs=[_X_SPEC, _P_SPEC, _X_SPEC, _D_SPEC,
                  pl.BlockSpec((128,), lambda i: (0,)),
                  pl.BlockSpec((128, 128), lambda i: (0, 0))],
        out_specs=_X_SPEC,
        out_shape=jax.ShapeDtypeStruct((NP, 128), jnp.float32),
    )(acc, p, xj, dinv, b3j, W4j)


def _tc_d2_body(acc_ref, dinv_ref, out_ref):
    out_ref[...] = acc_ref[...] * dinv_ref[...]


def _tc_d2(acc4, dinv):
    return pl.pallas_call(
        _tc_d2_body,
        grid=(GRID,),
        in_specs=[_X_SPEC, _D_SPEC],
        out_specs=_X_SPEC,
        out_shape=jax.ShapeDtypeStruct((NP, 128), jnp.float32),
    )(acc4, dinv)


def _tc_e_body(p_ref, x0_ref, dinv_ref, b4_ref, out_ref):
    g = _sum_partials(p_ref, 1, [x0_ref])
    out_ref[...] = g * dinv_ref[...] + b4_ref[...]


def _tc_e(p4, t4, dinv, b4):
    return pl.pallas_call(
        _tc_e_body,
        grid=(GRID,),
        in_specs=[
            pl.BlockSpec((SC_CORES, 1, ROW_BLK, 128), lambda i: (0, 0, i, 0)),
            pl.BlockSpec((ROW_BLK, 128), lambda i: (i, 0)),
            pl.BlockSpec((ROW_BLK, 1), lambda i: (i, 0)),
            pl.BlockSpec((128,), lambda i: (0,)),
        ],
        out_specs=pl.BlockSpec((ROW_BLK, 128), lambda i: (i, 0)),
        out_shape=jax.ShapeDtypeStruct((NP, 128), jnp.float32),
    )(p4, t4, dinv, b4)


@jax.jit
def _run(x, edge_index, W1, b1, W2, b2, W3, b3, W4, b4):
    ei = edge_index.astype(jnp.int32)
    pad = jnp.arange(PADN, dtype=jnp.int32)
    src = jnp.concatenate([ei[0], pad]).reshape(NW, EPW)
    dst_all = jnp.concatenate([ei[1], pad])
    dst = dst_all.reshape(NW, BPW, BLK)
    ztile = jnp.zeros((RPT, 128), jnp.float32)

    x_pad = jnp.pad(x, ((0, NP - N), (0, 0)))
    degp = _sc_deg(dst_all.reshape(NW, EPW))
    degp_t = degp.T.reshape(GRID, ROW_BLK, NW)
    dinv, x1s = _tc_a(degp_t, x_pad)
    p1 = _sc_spmm1(src, dst, ztile, x1s)
    h1s = _tc_b(p1, x1s, dinv, W1, b1)
    a2 = None
    for j in range(4):
        p2j = _sc_spmm1(src, dst, ztile, h1s[j])
        Wj = W2[j * 128:(j + 1) * 128]
        if a2 is None:
            a2 = _tc_l2_init(p2j, h1s[j], dinv, Wj, b2)
        else:
            a2 = _tc_l2_step(a2, p2j, h1s[j], dinv, Wj)
    t3 = _tc_c2(a2, dinv, W3)
    a4 = None
    for j in range(4):
        p3j = _sc_spmm1(src, dst, ztile, t3[j])
        b3j = b3[j * 128:(j + 1) * 128]
        W4j = W4[j * 128:(j + 1) * 128]
        if a4 is None:
            a4 = _tc_l3_init(p3j, t3[j], dinv, b3j, W4j)
        else:
            a4 = _tc_l3_step(a4, p3j, t3[j], dinv, b3j, W4j)
    t4 = _tc_d2(a4, dinv)
    p4 = _sc_spmm1(src, dst, ztile, t4)
    out = _tc_e(p4, t4, dinv, b4)
    return out[:N]


def kernel(x, edge_index, W1, b1, W2, b2, W3, b3, W4, b4):
    return _run(x, edge_index, W1, b1, W2, b2, W3, b3, W4, b4)


# speedup vs baseline: 1.1086x; 1.1086x over previous
import functools

import jax
import jax.numpy as jnp
from jax import lax
from jax.experimental import pallas as pl
from jax.experimental.pallas import tpu as pltpu
from jax.experimental.pallas import tpu_sc as plsc

N = 10000
E = 320000
SC_CORES = 2
SC_SUBCORES = 16
LANES = 16
NW = SC_CORES * SC_SUBCORES
BLK = 96
BPW = 105
EPW = BPW * BLK
E_PAD = NW * EPW
PADN = E_PAD - E
NBUF = 2
NP = 10112
RPT = NP // SC_SUBCORES
ROW_BLK = 632
GRID = NP // ROW_BLK

_MESH = plsc.VectorSubcoreMesh(core_axis_name="c", subcore_axis_name="s",
                               num_cores=SC_CORES, num_subcores=SC_SUBCORES)


def _deg_body(dst_hbm, out_hbm, dst_v, hist_v):
    c = lax.axis_index("c")
    s = lax.axis_index("s")
    wid = c * SC_SUBCORES + s
    pltpu.sync_copy(dst_hbm.at[wid], dst_v)
    zero16 = jnp.zeros((LANES,), jnp.float32)
    ones16 = jnp.ones((LANES,), jnp.float32)

    def zb(i, carry):
        hist_v[pl.ds(i * LANES, LANES)] = zero16
        return carry

    lax.fori_loop(0, NP // LANES, zb, 0)

    def hb(i, carry):
        idx = dst_v[pl.ds(i * LANES, LANES)]
        plsc.addupdate_scatter(hist_v, [idx], ones16)
        return carry

    lax.fori_loop(0, EPW // LANES, hb, 0)
    pltpu.sync_copy(hist_v, out_hbm.at[wid])


_SC_PARAMS = pltpu.CompilerParams(needs_layout_passes=False)

_sc_deg = pl.kernel(
    _deg_body,
    out_type=jax.ShapeDtypeStruct((NW, NP), jnp.float32),
    mesh=_MESH,
    compiler_params=_SC_PARAMS,
    scratch_types=[
        pltpu.VMEM((EPW,), jnp.int32),
        pltpu.VMEM((NP,), jnp.float32),
    ],
)


def _make_spmm(nc):
    def body(src_hbm, dst_hbm, zeros_hbm, *rest):
        xs = rest[:nc]
        out_hbm = rest[nc]
        src_v, dst_v, gb0, gb1, acc = rest[nc + 1:nc + 6]
        gsem0, gsem1 = rest[nc + 6:nc + 8]
        gb = (gb0, gb1)
        gsem = (gsem0, gsem1)
        c = lax.axis_index("c")
        s = lax.axis_index("s")
        wid = c * SC_SUBCORES + s
        pltpu.async_copy(src_hbm.at[wid], src_v, gsem0)
        pltpu.async_copy(dst_hbm.at[wid], dst_v, gsem1)
        idx_pending = True

        for j in range(nc):
            xs_j = xs[j]
            pltpu.sync_copy(zeros_hbm, acc.at[pl.ds(s * RPT, RPT), :])
            if idx_pending:
                pltpu.make_async_copy(src_hbm.at[wid], src_v, gsem0).wait()
                pltpu.make_async_copy(dst_hbm.at[wid], dst_v, gsem1).wait()
                idx_pending = False
            plsc.subcore_barrier()

            for k in range(NBUF):
                pltpu.async_copy(xs_j.at[src_v.at[pl.ds(k * BLK, BLK)]],
                                 gb[k], gsem[k])

            def rnd(r, carry):
                for k in range(NBUF):
                    b = r * NBUF + k
                    pltpu.make_async_copy(
                        xs_j.at[src_v.at[pl.ds(b * BLK, BLK)]], gb[k],
                        gsem[k]).wait()
                    pltpu.sync_copy(gb[k], acc.at[dst_v.at[b]], add=True)
                    pltpu.async_copy(
                        xs_j.at[src_v.at[pl.ds((b + NBUF) * BLK, BLK)]],
                        gb[k], gsem[k])
                return carry

            lax.fori_loop(0, BPW // NBUF - 1, rnd, 0)
            for b in range((BPW // NBUF - 1) * NBUF, BPW):
                k = b % NBUF
                pltpu.make_async_copy(
                    xs_j.at[src_v.at[pl.ds(b * BLK, BLK)]], gb[k],
                    gsem[k]).wait()
                pltpu.sync_copy(gb[k], acc.at[dst_v.at[b]], add=True)
                if b + NBUF < BPW:
                    pltpu.async_copy(
                        xs_j.at[src_v.at[pl.ds((b + NBUF) * BLK, BLK)]],
                        gb[k], gsem[k])
            plsc.subcore_barrier()
            pltpu.sync_copy(acc.at[pl.ds(s * RPT, RPT), :],
                            out_hbm.at[c, j, pl.ds(s * RPT, RPT), :])

    return pl.kernel(
        body,
        out_type=jax.ShapeDtypeStruct((SC_CORES, nc, NP, 128), jnp.float32),
        mesh=_MESH,
        compiler_params=_SC_PARAMS,
        scratch_types=(
            [pltpu.VMEM((EPW,), jnp.int32),
             pltpu.VMEM((BPW, BLK), jnp.int32)]
            + [pltpu.VMEM((BLK, 128), jnp.float32)] * 2
            + [pltpu.VMEM_SHARED((NP, 128), jnp.float32)]
            + [pltpu.SemaphoreType.DMA] * 2
        ),
    )


_sc_spmm1 = _make_spmm(1)
_sc_spmm4 = _make_spmm(4)


def _gelu(x):
    return 0.5 * x * (1.0 + lax.erf(x * 0.7071067811865476))


def _id_mask():
    rid = (lax.broadcasted_iota(jnp.int32, (ROW_BLK, 1), 0)
           + pl.program_id(0) * ROW_BLK)
    return ((rid >= PADN) & (rid < N)).astype(jnp.float32)


def _tc_a_body(degp_ref, x_ref, dinv_ref, x1s_ref):
    deg = jnp.sum(degp_ref[0], axis=1) + _id_mask()[:, 0]
    dinv = jnp.where(deg > 0, lax.rsqrt(jnp.maximum(deg, 1e-12)), 0.0)
    dinv_ref[...] = dinv[:, None]
    x1s_ref[...] = x_ref[...] * dinv[:, None]


def _tc_a(degp, x_pad):
    return pl.pallas_call(
        _tc_a_body,
        grid=(GRID,),
        in_specs=[
            pl.BlockSpec((1, ROW_BLK, NW), lambda i: (i, 0, 0)),
            pl.BlockSpec((ROW_BLK, 128), lambda i: (i, 0)),
        ],
        out_specs=[
            pl.BlockSpec((ROW_BLK, 1), lambda i: (i, 0)),
            pl.BlockSpec((ROW_BLK, 128), lambda i: (i, 0)),
        ],
        out_shape=[
            jax.ShapeDtypeStruct((NP, 1), jnp.float32),
            jax.ShapeDtypeStruct((NP, 128), jnp.float32),
        ],
    )(degp, x_pad)


def _sum_partials(p_ref, nc, x_refs):
    m = _id_mask()
    g = p_ref[0] + p_ref[1]
    return jnp.concatenate(
        [g[j] + m * x_refs[j][...] for j in range(nc)], axis=1)


def _tc_b_body(p_ref, x0_ref, dinv_ref, w1_ref, b1_ref, *out_refs):
    g = _sum_partials(p_ref, 1, [x0_ref])
    dinv = dinv_ref[...]
    h = jnp.dot((g * dinv).astype(jnp.bfloat16),
                w1_ref[...].astype(jnp.bfloat16),
                preferred_element_type=jnp.float32) + b1_ref[...]
    h = _gelu(h) * dinv
    for j in range(4):
        out_refs[j][...] = h[:, j * 128:(j + 1) * 128]


def _tc_b(p1, x1s, dinv, W1, b1):
    return pl.pallas_call(
        _tc_b_body,
        grid=(GRID,),
        in_specs=[
            pl.BlockSpec((SC_CORES, 1, ROW_BLK, 128), lambda i: (0, 0, i, 0)),
            pl.BlockSpec((ROW_BLK, 128), lambda i: (i, 0)),
            pl.BlockSpec((ROW_BLK, 1), lambda i: (i, 0)),
            pl.BlockSpec((128, 512), lambda i: (0, 0)),
            pl.BlockSpec((512,), lambda i: (0,)),
        ],
        out_specs=[pl.BlockSpec((ROW_BLK, 128), lambda i: (i, 0))] * 4,
        out_shape=[jax.ShapeDtypeStruct((NP, 128), jnp.float32)] * 4,
    )(p1, x1s, dinv, W1, b1)


def _tc_c_body(p_ref, x0, x1, x2, x3, dinv_ref, w2_ref, b2_ref, w3_ref,
               *out_refs):
    g = _sum_partials(p_ref, 4, [x0, x1, x2, x3])
    dinv = dinv_ref[...]
    h = jnp.dot((g * dinv).astype(jnp.bfloat16),
                w2_ref[...].astype(jnp.bfloat16),
                preferred_element_type=jnp.float32) + b2_ref[...]
    h = _gelu(h)
    t = jnp.dot(h.astype(jnp.bfloat16), w3_ref[...].astype(jnp.bfloat16),
                preferred_element_type=jnp.float32)
    t = t * dinv
    for j in range(4):
        out_refs[j][...] = t[:, j * 128:(j + 1) * 128]


def _tc_c(p2, xs, dinv, W2, b2, W3):
    return pl.pallas_call(
        _tc_c_body,
        grid=(GRID,),
        in_specs=[
            pl.BlockSpec((SC_CORES, 4, ROW_BLK, 128), lambda i: (0, 0, i, 0)),
        ] + [pl.BlockSpec((ROW_BLK, 128), lambda i: (i, 0))] * 4 + [
            pl.BlockSpec((ROW_BLK, 1), lambda i: (i, 0)),
            pl.BlockSpec((512, 768), lambda i: (0, 0)),
            pl.BlockSpec((768,), lambda i: (0,)),
            pl.BlockSpec((768, 512), lambda i: (0, 0)),
        ],
        out_specs=[pl.BlockSpec((ROW_BLK, 128), lambda i: (i, 0))] * 4,
        out_shape=[jax.ShapeDtypeStruct((NP, 128), jnp.float32)] * 4,
    )(p2, *xs, dinv, W2, b2, W3)


def _tc_d_body(p_ref, x0, x1, x2, x3, dinv_ref, b3_ref, w4_ref, out_ref):
    g = _sum_partials(p_ref, 4, [x0, x1, x2, x3])
    dinv = dinv_ref[...]
    h = _gelu(g * dinv + b3_ref[...])
    t = jnp.dot(h.astype(jnp.bfloat16), w4_ref[...].astype(jnp.bfloat16),
                preferred_element_type=jnp.float32)
    out_ref[...] = t * dinv


def _tc_d(p3, xs, dinv, b3, W4):
    return pl.pallas_call(
        _tc_d_body,
        grid=(GRID,),
        in_specs=[
            pl.BlockSpec((SC_CORES, 4, ROW_BLK, 128), lambda i: (0, 0, i, 0)),
        ] + [pl.BlockSpec((ROW_BLK, 128), lambda i: (i, 0))] * 4 + [
            pl.BlockSpec((ROW_BLK, 1), lambda i: (i, 0)),
            pl.BlockSpec((512,), lambda i: (0,)),
            pl.BlockSpec((512, 128), lambda i: (0, 0)),
        ],
        out_specs=pl.BlockSpec((ROW_BLK, 128), lambda i: (i, 0)),
        out_shape=jax.ShapeDtypeStruct((NP, 128), jnp.float32),
    )(p3, *xs, dinv, b3, W4)


def _tc_e_body(p_ref, x0_ref, dinv_ref, b4_ref, out_ref):
    g = _sum_partials(p_ref, 1, [x0_ref])
    out_ref[...] = g * dinv_ref[...] + b4_ref[...]


def _tc_e(p4, t4, dinv, b4):
    return pl.pallas_call(
        _tc_e_body,
        grid=(GRID,),
        in_specs=[
            pl.BlockSpec((SC_CORES, 1, ROW_BLK, 128), lambda i: (0, 0, i, 0)),
            pl.BlockSpec((ROW_BLK, 128), lambda i: (i, 0)),
            pl.BlockSpec((ROW_BLK, 1), lambda i: (i, 0)),
            pl.BlockSpec((128,), lambda i: (0,)),
        ],
        out_specs=pl.BlockSpec((ROW_BLK, 128), lambda i: (i, 0)),
        out_shape=jax.ShapeDtypeStruct((NP, 128), jnp.float32),
    )(p4, t4, dinv, b4)


@jax.jit
def _run(x, edge_index, W1, b1, W2, b2, W3, b3, W4, b4):
    ei = edge_index.astype(jnp.int32)
    pad = jnp.arange(PADN, dtype=jnp.int32)
    src = jnp.concatenate([ei[0], pad]).reshape(NW, EPW)
    dst_all = jnp.concatenate([ei[1], pad])
    dst = dst_all.reshape(NW, BPW, BLK)
    ztile = jnp.zeros((RPT, 128), jnp.float32)

    x_pad = jnp.pad(x, ((0, NP - N), (0, 0)))
    degp = _sc_deg(dst_all.reshape(NW, EPW))
    degp_t = degp.T.reshape(GRID, ROW_BLK, NW)
    dinv, x1s = _tc_a(degp_t, x_pad)
    p1 = _sc_spmm1(src, dst, ztile, x1s)
    h1s = _tc_b(p1, x1s, dinv, W1, b1)
    p2 = _sc_spmm4(src, dst, ztile, *h1s)
    t3 = _tc_c(p2, h1s, dinv, W2, b2, W3)
    p3 = _sc_spmm4(src, dst, ztile, *t3)
    t4 = _tc_d(p3, t3, dinv, b3, W4)
    p4 = _sc_spmm1(src, dst, ztile, t4)
    out = _tc_e(p4, t4, dinv, b4)
    return out[:N]


def kernel(x, edge_index, W1, b1, W2, b2, W3, b3, W4, b4):
    return _run(x, edge_index, W1, b1, W2, b2, W3, b3, W4, b4)


# speedup vs baseline: 1.1093x; 1.0006x over previous
import jax
import jax.numpy as jnp
from jax import lax
from jax.experimental import pallas as pl
from jax.experimental.pallas import tpu as pltpu
from jax.experimental.pallas import tpu_sc as plsc

N = 10000
E = 320000
SC_CORES = 2
SC_SUBCORES = 16
LANES = 16
NW = SC_CORES * SC_SUBCORES
BLK = 96
BPW = 105
EPW = BPW * BLK
E_PAD = NW * EPW
PADN = E_PAD - E
NBUF = 2
NP = 10112
RPT = NP // SC_SUBCORES
ROW_BLK = 632
GRID = NP // ROW_BLK

_MESH = plsc.VectorSubcoreMesh(core_axis_name="c", subcore_axis_name="s",
                               num_cores=SC_CORES, num_subcores=SC_SUBCORES)


def _deg_body(dst_hbm, out_hbm, dst_v, hist_v):
    c = lax.axis_index("c")
    s = lax.axis_index("s")
    wid = c * SC_SUBCORES + s
    pltpu.sync_copy(dst_hbm.at[wid], dst_v)
    zero16 = jnp.zeros((LANES,), jnp.float32)
    ones16 = jnp.ones((LANES,), jnp.float32)

    def zb(i, carry):
        hist_v[pl.ds(i * LANES, LANES)] = zero16
        return carry

    lax.fori_loop(0, NP // LANES, zb, 0)

    def hb(i, carry):
        idx = dst_v[pl.ds(i * LANES, LANES)]
        plsc.addupdate_scatter(hist_v, [idx], ones16)
        return carry

    lax.fori_loop(0, EPW // LANES, hb, 0)
    pltpu.sync_copy(hist_v, out_hbm.at[wid])


_SC_PARAMS = pltpu.CompilerParams(needs_layout_passes=False)

_sc_deg = pl.kernel(
    _deg_body,
    out_type=jax.ShapeDtypeStruct((NW, NP), jnp.float32),
    mesh=_MESH,
    compiler_params=_SC_PARAMS,
    scratch_types=[
        pltpu.VMEM((EPW,), jnp.int32),
        pltpu.VMEM((NP,), jnp.float32),
    ],
)


def _make_spmm(nc):
    def body(src_hbm, dst_hbm, zeros_hbm, *rest):
        xs = rest[:nc]
        out_hbm = rest[nc]
        src_v, dst_v, gb0, gb1, acc = rest[nc + 1:nc + 6]
        gsem0, gsem1 = rest[nc + 6:nc + 8]
        gb = (gb0, gb1)
        gsem = (gsem0, gsem1)
        c = lax.axis_index("c")
        s = lax.axis_index("s")
        wid = c * SC_SUBCORES + s
        pltpu.async_copy(src_hbm.at[wid], src_v, gsem0)
        pltpu.async_copy(dst_hbm.at[wid], dst_v, gsem1)
        idx_pending = True

        for j in range(nc):
            xs_j = xs[j]
            pltpu.sync_copy(zeros_hbm, acc.at[pl.ds(s * RPT, RPT), :])
            if idx_pending:
                pltpu.make_async_copy(src_hbm.at[wid], src_v, gsem0).wait()
                pltpu.make_async_copy(dst_hbm.at[wid], dst_v, gsem1).wait()
                idx_pending = False
            plsc.subcore_barrier()

            for k in range(NBUF):
                pltpu.async_copy(xs_j.at[src_v.at[pl.ds(k * BLK, BLK)]],
                                 gb[k], gsem[k])

            def rnd(r, carry):
                for k in range(NBUF):
                    b = r * NBUF + k
                    pltpu.make_async_copy(
                        xs_j.at[src_v.at[pl.ds(b * BLK, BLK)]], gb[k],
                        gsem[k]).wait()
                    pltpu.sync_copy(gb[k], acc.at[dst_v.at[b]], add=True)
                    pltpu.async_copy(
                        xs_j.at[src_v.at[pl.ds((b + NBUF) * BLK, BLK)]],
                        gb[k], gsem[k])
                return carry

            lax.fori_loop(0, BPW // NBUF - 1, rnd, 0)
            for b in range((BPW // NBUF - 1) * NBUF, BPW):
                k = b % NBUF
                pltpu.make_async_copy(
                    xs_j.at[src_v.at[pl.ds(b * BLK, BLK)]], gb[k],
                    gsem[k]).wait()
                pltpu.sync_copy(gb[k], acc.at[dst_v.at[b]], add=True)
                if b + NBUF < BPW:
                    pltpu.async_copy(
                        xs_j.at[src_v.at[pl.ds((b + NBUF) * BLK, BLK)]],
                        gb[k], gsem[k])
            plsc.subcore_barrier()
            pltpu.sync_copy(acc.at[pl.ds(s * RPT, RPT), :],
                            out_hbm.at[c, j, pl.ds(s * RPT, RPT), :])

    return pl.kernel(
        body,
        out_type=jax.ShapeDtypeStruct((SC_CORES, nc, NP, 128), jnp.float32),
        mesh=_MESH,
        compiler_params=_SC_PARAMS,
        scratch_types=(
            [pltpu.VMEM((EPW,), jnp.int32),
             pltpu.VMEM((BPW, BLK), jnp.int32)]
            + [pltpu.VMEM((BLK, 128), jnp.float32)] * 2
            + [pltpu.VMEM_SHARED((NP, 128), jnp.float32)]
            + [pltpu.SemaphoreType.DMA] * 2
        ),
    )


_sc_spmm1 = _make_spmm(1)
_sc_spmm4 = _make_spmm(4)


def _gelu(x):
    return 0.5 * x * (1.0 + lax.erf(x * 0.7071067811865476))


def _id_mask():
    rid = (lax.broadcasted_iota(jnp.int32, (ROW_BLK, 1), 0)
           + pl.program_id(0) * ROW_BLK)
    return ((rid >= PADN) & (rid < N)).astype(jnp.float32)


def _tc_a_body(degp_ref, x_ref, dinv_ref, x1s_ref):
    deg = jnp.sum(degp_ref[0], axis=1) + _id_mask()[:, 0]
    dinv = jnp.where(deg > 0, lax.rsqrt(jnp.maximum(deg, 1e-12)), 0.0)
    dinv_ref[...] = dinv[:, None]
    x1s_ref[...] = x_ref[...] * dinv[:, None]


def _tc_a(degp, x_pad):
    return pl.pallas_call(
        _tc_a_body,
        grid=(GRID,),
        in_specs=[
            pl.BlockSpec((1, ROW_BLK, NW), lambda i: (i, 0, 0)),
            pl.BlockSpec((ROW_BLK, 128), lambda i: (i, 0)),
        ],
        out_specs=[
            pl.BlockSpec((ROW_BLK, 1), lambda i: (i, 0)),
            pl.BlockSpec((ROW_BLK, 128), lambda i: (i, 0)),
        ],
        out_shape=[
            jax.ShapeDtypeStruct((NP, 1), jnp.float32),
            jax.ShapeDtypeStruct((NP, 128), jnp.float32),
        ],
    )(degp, x_pad)


def _sum_partials(p_ref, nc, x_refs):
    m = _id_mask()
    g = p_ref[0] + p_ref[1]
    return jnp.concatenate(
        [g[j] + m * x_refs[j][...] for j in range(nc)], axis=1)


def _tc_b_body(p_ref, x0_ref, dinv_ref, w1_ref, b1_ref, *out_refs):
    g = _sum_partials(p_ref, 1, [x0_ref])
    dinv = dinv_ref[...]
    h = jnp.dot(g * dinv, w1_ref[...],
                preferred_element_type=jnp.float32) + b1_ref[...]
    h = _gelu(h) * dinv
    for j in range(4):
        out_refs[j][...] = h[:, j * 128:(j + 1) * 128]


def _tc_b(p1, x1s, dinv, W1, b1):
    return pl.pallas_call(
        _tc_b_body,
        grid=(GRID,),
        in_specs=[
            pl.BlockSpec((SC_CORES, 1, ROW_BLK, 128), lambda i: (0, 0, i, 0)),
            pl.BlockSpec((ROW_BLK, 128), lambda i: (i, 0)),
            pl.BlockSpec((ROW_BLK, 1), lambda i: (i, 0)),
            pl.BlockSpec((128, 512), lambda i: (0, 0)),
            pl.BlockSpec((512,), lambda i: (0,)),
        ],
        out_specs=[pl.BlockSpec((ROW_BLK, 128), lambda i: (i, 0))] * 4,
        out_shape=[jax.ShapeDtypeStruct((NP, 128), jnp.float32)] * 4,
    )(p1, x1s, dinv, W1, b1)


def _tc_c_body(p_ref, x0, x1, x2, x3, dinv_ref, w2_ref, b2_ref, w3_ref,
               *out_refs):
    g = _sum_partials(p_ref, 4, [x0, x1, x2, x3])
    dinv = dinv_ref[...]
    h = jnp.dot(g * dinv, w2_ref[...],
                preferred_element_type=jnp.float32) + b2_ref[...]
    h = _gelu(h)
    t = jnp.dot(h, w3_ref[...], preferred_element_type=jnp.float32)
    t = t * dinv
    for j in range(4):
        out_refs[j][...] = t[:, j * 128:(j + 1) * 128]


def _tc_c(p2, xs, dinv, W2, b2, W3):
    return pl.pallas_call(
        _tc_c_body,
        grid=(GRID,),
        in_specs=[
            pl.BlockSpec((SC_CORES, 4, ROW_BLK, 128), lambda i: (0, 0, i, 0)),
        ] + [pl.BlockSpec((ROW_BLK, 128), lambda i: (i, 0))] * 4 + [
            pl.BlockSpec((ROW_BLK, 1), lambda i: (i, 0)),
            pl.BlockSpec((512, 768), lambda i: (0, 0)),
            pl.BlockSpec((768,), lambda i: (0,)),
            pl.BlockSpec((768, 512), lambda i: (0, 0)),
        ],
        out_specs=[pl.BlockSpec((ROW_BLK, 128), lambda i: (i, 0))] * 4,
        out_shape=[jax.ShapeDtypeStruct((NP, 128), jnp.float32)] * 4,
    )(p2, *xs, dinv, W2, b2, W3)


def _tc_d_body(p_ref, x0, x1, x2, x3, dinv_ref, b3_ref, w4_ref, out_ref):
    g = _sum_partials(p_ref, 4, [x0, x1, x2, x3])
    dinv = dinv_ref[...]
    h = _gelu(g * dinv + b3_ref[...])
    t = jnp.dot(h, w4_ref[...], preferred_element_type=jnp.float32)
    out_ref[...] = t * dinv


def _tc_d(p3, xs, dinv, b3, W4):
    return pl.pallas_call(
        _tc_d_body,
        grid=(GRID,),
        in_specs=[
            pl.BlockSpec((SC_CORES, 4, ROW_BLK, 128), lambda i: (0, 0, i, 0)),
        ] + [pl.BlockSpec((ROW_BLK, 128), lambda i: (i, 0))] * 4 + [
            pl.BlockSpec((ROW_BLK, 1), lambda i: (i, 0)),
            pl.BlockSpec((512,), lambda i: (0,)),
            pl.BlockSpec((512, 128), lambda i: (0, 0)),
        ],
        out_specs=pl.BlockSpec((ROW_BLK, 128), lambda i: (i, 0)),
        out_shape=jax.ShapeDtypeStruct((NP, 128), jnp.float32),
    )(p3, *xs, dinv, b3, W4)


def _tc_e_body(p_ref, x0_ref, dinv_ref, b4_ref, out_ref):
    g = _sum_partials(p_ref, 1, [x0_ref])
    out_ref[...] = g * dinv_ref[...] + b4_ref[...]


def _tc_e(p4, t4, dinv, b4):
    return pl.pallas_call(
        _tc_e_body,
        grid=(GRID,),
        in_specs=[
            pl.BlockSpec((SC_CORES, 1, ROW_BLK, 128), lambda i: (0, 0, i, 0)),
            pl.BlockSpec((ROW_BLK, 128), lambda i: (i, 0)),
            pl.BlockSpec((ROW_BLK, 1), lambda i: (i, 0)),
            pl.BlockSpec((128,), lambda i: (0,)),
        ],
        out_specs=pl.BlockSpec((ROW_BLK, 128), lambda i: (i, 0)),
        out_shape=jax.ShapeDtypeStruct((NP, 128), jnp.float32),
    )(p4, t4, dinv, b4)


@jax.jit
def _run(x, edge_index, W1, b1, W2, b2, W3, b3, W4, b4):
    ei = edge_index.astype(jnp.int32)
    pad = jnp.arange(PADN, dtype=jnp.int32)
    src = jnp.concatenate([ei[0], pad]).reshape(NW, EPW)
    dst_all = jnp.concatenate([ei[1], pad])
    dst = dst_all.reshape(NW, BPW, BLK)
    ztile = jnp.zeros((RPT, 128), jnp.float32)

    x_pad = jnp.pad(x, ((0, NP - N), (0, 0)))
    degp = _sc_deg(dst_all.reshape(NW, EPW))
    degp_t = degp.T.reshape(GRID, ROW_BLK, NW)
    dinv, x1s = _tc_a(degp_t, x_pad)
    p1 = _sc_spmm1(src, dst, ztile, x1s)
    h1s = _tc_b(p1, x1s, dinv, W1, b1)
    p2 = _sc_spmm4(src, dst, ztile, *h1s)
    t3 = _tc_c(p2, h1s, dinv, W2, b2, W3)
    p3 = _sc_spmm4(src, dst, ztile, *t3)
    t4 = _tc_d(p3, t3, dinv, b3, W4)
    p4 = _sc_spmm1(src, dst, ztile, t4)
    out = _tc_e(p4, t4, dinv, b4)
    return out[:N]


def kernel(x, edge_index, W1, b1, W2, b2, W3, b3, W4, b4):
    return _run(x, edge_index, W1, b1, W2, b2, W3, b3, W4, b4)


# speedup vs baseline: 1.1136x; 1.0039x over previous
import jax
import jax.numpy as jnp
from jax import lax
from jax.experimental import pallas as pl
from jax.experimental.pallas import tpu as pltpu
from jax.experimental.pallas import tpu_sc as plsc

N = 10000
E = 320000
SC_CORES = 2
SC_SUBCORES = 16
LANES = 16
NW = SC_CORES * SC_SUBCORES
BLK = 96
BPW = 105
EPW = BPW * BLK
E_PAD = NW * EPW
PADN = E_PAD - E
NBUF = 2
NP = 10112
RPT = NP // SC_SUBCORES
ROW_BLK = 632
GRID = NP // ROW_BLK

_MESH = plsc.VectorSubcoreMesh(core_axis_name="c", subcore_axis_name="s",
                               num_cores=SC_CORES, num_subcores=SC_SUBCORES)


def _deg_body(dst_hbm, out_hbm, dst_v, hist_v):
    c = lax.axis_index("c")
    s = lax.axis_index("s")
    wid = c * SC_SUBCORES + s
    pltpu.sync_copy(dst_hbm.at[wid], dst_v)
    zero16 = jnp.zeros((LANES,), jnp.float32)
    ones16 = jnp.ones((LANES,), jnp.float32)

    def zb(i, carry):
        hist_v[pl.ds(i * LANES, LANES)] = zero16
        return carry

    lax.fori_loop(0, NP // LANES, zb, 0)

    def hb(i, carry):
        idx = dst_v[pl.ds(i * LANES, LANES)]
        plsc.addupdate_scatter(hist_v, [idx], ones16)
        return carry

    lax.fori_loop(0, EPW // LANES, hb, 0)
    pltpu.sync_copy(hist_v, out_hbm.at[wid])


_SC_PARAMS = pltpu.CompilerParams(needs_layout_passes=False)

_sc_deg = pl.kernel(
    _deg_body,
    out_type=jax.ShapeDtypeStruct((NW, NP), jnp.float32),
    mesh=_MESH,
    compiler_params=_SC_PARAMS,
    scratch_types=[
        pltpu.VMEM((EPW,), jnp.int32),
        pltpu.VMEM((NP,), jnp.float32),
    ],
)


def _make_spmm(nc):
    def body(src_hbm, dst_hbm, zeros_hbm, *rest):
        xs = rest[:nc]
        out_hbm = rest[nc]
        src_v, dst_v, gb0, gb1, acc = rest[nc + 1:nc + 6]
        gsem0, gsem1 = rest[nc + 6:nc + 8]
        gb = (gb0, gb1)
        gsem = (gsem0, gsem1)
        c = lax.axis_index("c")
        s = lax.axis_index("s")
        wid = c * SC_SUBCORES + s
        pltpu.async_copy(src_hbm.at[wid], src_v, gsem0)
        pltpu.async_copy(dst_hbm.at[wid], dst_v, gsem1)
        idx_pending = True

        def prime(xs_j):
            for k in range(NBUF):
                pltpu.async_copy(xs_j.at[src_v.at[pl.ds(k * BLK, BLK)]],
                                 gb[k], gsem[k])

        for j in range(nc):
            xs_j = xs[j]
            if idx_pending:
                pltpu.sync_copy(zeros_hbm, acc.at[pl.ds(s * RPT, RPT), :])
                pltpu.make_async_copy(src_hbm.at[wid], src_v, gsem0).wait()
                pltpu.make_async_copy(dst_hbm.at[wid], dst_v, gsem1).wait()
                idx_pending = False
                prime(xs_j)
            else:
                prime(xs_j)
                pltpu.sync_copy(zeros_hbm, acc.at[pl.ds(s * RPT, RPT), :])
            plsc.subcore_barrier()

            def rnd(r, carry):
                for k in range(NBUF):
                    b = r * NBUF + k
                    pltpu.make_async_copy(
                        xs_j.at[src_v.at[pl.ds(b * BLK, BLK)]], gb[k],
                        gsem[k]).wait()
                    pltpu.sync_copy(gb[k], acc.at[dst_v.at[b]], add=True)
                    pltpu.async_copy(
                        xs_j.at[src_v.at[pl.ds((b + NBUF) * BLK, BLK)]],
                        gb[k], gsem[k])
                return carry

            lax.fori_loop(0, BPW // NBUF - 1, rnd, 0)
            for b in range((BPW // NBUF - 1) * NBUF, BPW):
                k = b % NBUF
                pltpu.make_async_copy(
                    xs_j.at[src_v.at[pl.ds(b * BLK, BLK)]], gb[k],
                    gsem[k]).wait()
                pltpu.sync_copy(gb[k], acc.at[dst_v.at[b]], add=True)
                if b + NBUF < BPW:
                    pltpu.async_copy(
                        xs_j.at[src_v.at[pl.ds((b + NBUF) * BLK, BLK)]],
                        gb[k], gsem[k])
            plsc.subcore_barrier()
            pltpu.sync_copy(acc.at[pl.ds(s * RPT, RPT), :],
                            out_hbm.at[c, j, pl.ds(s * RPT, RPT), :])

    return pl.kernel(
        body,
        out_type=jax.ShapeDtypeStruct((SC_CORES, nc, NP, 128), jnp.float32),
        mesh=_MESH,
        compiler_params=_SC_PARAMS,
        scratch_types=(
            [pltpu.VMEM((EPW,), jnp.int32),
             pltpu.VMEM((BPW, BLK), jnp.int32)]
            + [pltpu.VMEM((BLK, 128), jnp.float32)] * 2
            + [pltpu.VMEM_SHARED((NP, 128), jnp.float32)]
            + [pltpu.SemaphoreType.DMA] * 2
        ),
    )


_sc_spmm1 = _make_spmm(1)
_sc_spmm4 = _make_spmm(4)


def _gelu(x):
    return 0.5 * x * (1.0 + lax.erf(x * 0.7071067811865476))


def _id_mask():
    rid = (lax.broadcasted_iota(jnp.int32, (ROW_BLK, 1), 0)
           + pl.program_id(0) * ROW_BLK)
    return ((rid >= PADN) & (rid < N)).astype(jnp.float32)


def _tc_a_body(degp_ref, x_ref, dinv_ref, x1s_ref):
    deg = jnp.sum(degp_ref[0], axis=1) + _id_mask()[:, 0]
    dinv = jnp.where(deg > 0, lax.rsqrt(jnp.maximum(deg, 1e-12)), 0.0)
    dinv_ref[...] = dinv[:, None]
    x1s_ref[...] = x_ref[...] * dinv[:, None]


def _tc_a(degp, x_pad):
    return pl.pallas_call(
        _tc_a_body,
        grid=(GRID,),
        in_specs=[
            pl.BlockSpec((1, ROW_BLK, NW), lambda i: (i, 0, 0)),
            pl.BlockSpec((ROW_BLK, 128), lambda i: (i, 0)),
        ],
        out_specs=[
            pl.BlockSpec((ROW_BLK, 1), lambda i: (i, 0)),
            pl.BlockSpec((ROW_BLK, 128), lambda i: (i, 0)),
        ],
        out_shape=[
            jax.ShapeDtypeStruct((NP, 1), jnp.float32),
            jax.ShapeDtypeStruct((NP, 128), jnp.float32),
        ],
    )(degp, x_pad)


def _sum_partials(p_ref, nc, x_refs):
    m = _id_mask()
    g = p_ref[0] + p_ref[1]
    return jnp.concatenate(
        [g[j] + m * x_refs[j][...] for j in range(nc)], axis=1)


def _tc_b_body(p_ref, x0_ref, dinv_ref, w1_ref, b1_ref, *out_refs):
    g = _sum_partials(p_ref, 1, [x0_ref])
    dinv = dinv_ref[...]
    h = jnp.dot(g * dinv, w1_ref[...],
                preferred_element_type=jnp.float32) + b1_ref[...]
    h = _gelu(h) * dinv
    for j in range(4):
        out_refs[j][...] = h[:, j * 128:(j + 1) * 128]


def _tc_b(p1, x1s, dinv, W1, b1):
    return pl.pallas_call(
        _tc_b_body,
        grid=(GRID,),
        in_specs=[
            pl.BlockSpec((SC_CORES, 1, ROW_BLK, 128), lambda i: (0, 0, i, 0)),
            pl.BlockSpec((ROW_BLK, 128), lambda i: (i, 0)),
            pl.BlockSpec((ROW_BLK, 1), lambda i: (i, 0)),
            pl.BlockSpec((128, 512), lambda i: (0, 0)),
            pl.BlockSpec((512,), lambda i: (0,)),
        ],
        out_specs=[pl.BlockSpec((ROW_BLK, 128), lambda i: (i, 0))] * 4,
        out_shape=[jax.ShapeDtypeStruct((NP, 128), jnp.float32)] * 4,
    )(p1, x1s, dinv, W1, b1)


def _tc_c_body(p_ref, x0, x1, x2, x3, dinv_ref, w2_ref, b2_ref, w3_ref,
               *out_refs):
    g = _sum_partials(p_ref, 4, [x0, x1, x2, x3])
    dinv = dinv_ref[...]
    h = jnp.dot(g * dinv, w2_ref[...],
                preferred_element_type=jnp.float32) + b2_ref[...]
    h = _gelu(h)
    t = jnp.dot(h, w3_ref[...], preferred_element_type=jnp.float32)
    t = t * dinv
    for j in range(4):
        out_refs[j][...] = t[:, j * 128:(j + 1) * 128]


def _tc_c(p2, xs, dinv, W2, b2, W3):
    return pl.pallas_call(
        _tc_c_body,
        grid=(GRID,),
        in_specs=[
            pl.BlockSpec((SC_CORES, 4, ROW_BLK, 128), lambda i: (0, 0, i, 0)),
        ] + [pl.BlockSpec((ROW_BLK, 128), lambda i: (i, 0))] * 4 + [
            pl.BlockSpec((ROW_BLK, 1), lambda i: (i, 0)),
            pl.BlockSpec((512, 768), lambda i: (0, 0)),
            pl.BlockSpec((768,), lambda i: (0,)),
            pl.BlockSpec((768, 512), lambda i: (0, 0)),
        ],
        out_specs=[pl.BlockSpec((ROW_BLK, 128), lambda i: (i, 0))] * 4,
        out_shape=[jax.ShapeDtypeStruct((NP, 128), jnp.float32)] * 4,
    )(p2, *xs, dinv, W2, b2, W3)


def _tc_d_body(p_ref, x0, x1, x2, x3, dinv_ref, b3_ref, w4_ref, out_ref):
    g = _sum_partials(p_ref, 4, [x0, x1, x2, x3])
    dinv = dinv_ref[...]
    h = _gelu(g * dinv + b3_ref[...])
    t = jnp.dot(h, w4_ref[...], preferred_element_type=jnp.float32)
    out_ref[...] = t * dinv


def _tc_d(p3, xs, dinv, b3, W4):
    return pl.pallas_call(
        _tc_d_body,
        grid=(GRID,),
        in_specs=[
            pl.BlockSpec((SC_CORES, 4, ROW_BLK, 128), lambda i: (0, 0, i, 0)),
        ] + [pl.BlockSpec((ROW_BLK, 128), lambda i: (i, 0))] * 4 + [
            pl.BlockSpec((ROW_BLK, 1), lambda i: (i, 0)),
            pl.BlockSpec((512,), lambda i: (0,)),
            pl.BlockSpec((512, 128), lambda i: (0, 0)),
        ],
        out_specs=pl.BlockSpec((ROW_BLK, 128), lambda i: (i, 0)),
        out_shape=jax.ShapeDtypeStruct((NP, 128), jnp.float32),
    )(p3, *xs, dinv, b3, W4)


def _tc_e_body(p_ref, x0_ref, dinv_ref, b4_ref, out_ref):
    g = _sum_partials(p_ref, 1, [x0_ref])
    out_ref[...] = g * dinv_ref[...] + b4_ref[...]


def _tc_e(p4, t4, dinv, b4):
    return pl.pallas_call(
        _tc_e_body,
        grid=(GRID,),
        in_specs=[
            pl.BlockSpec((SC_CORES, 1, ROW_BLK, 128), lambda i: (0, 0, i, 0)),
            pl.BlockSpec((ROW_BLK, 128), lambda i: (i, 0)),
            pl.BlockSpec((ROW_BLK, 1), lambda i: (i, 0)),
            pl.BlockSpec((128,), lambda i: (0,)),
        ],
        out_specs=pl.BlockSpec((ROW_BLK, 128), lambda i: (i, 0)),
        out_shape=jax.ShapeDtypeStruct((NP, 128), jnp.float32),
    )(p4, t4, dinv, b4)


@jax.jit
def _run(x, edge_index, W1, b1, W2, b2, W3, b3, W4, b4):
    ei = edge_index.astype(jnp.int32)
    pad = jnp.arange(PADN, dtype=jnp.int32)
    src = jnp.concatenate([ei[0], pad]).reshape(NW, EPW)
    dst_all = jnp.concatenate([ei[1], pad])
    dst = dst_all.reshape(NW, BPW, BLK)
    ztile = jnp.zeros((RPT, 128), jnp.float32)

    x_pad = jnp.pad(x, ((0, NP - N), (0, 0)))
    degp = _sc_deg(dst_all.reshape(NW, EPW))
    degp_t = degp.T.reshape(GRID, ROW_BLK, NW)
    dinv, x1s = _tc_a(degp_t, x_pad)
    p1 = _sc_spmm1(src, dst, ztile, x1s)
    h1s = _tc_b(p1, x1s, dinv, W1, b1)
    p2 = _sc_spmm4(src, dst, ztile, *h1s)
    t3 = _tc_c(p2, h1s, dinv, W2, b2, W3)
    p3 = _sc_spmm4(src, dst, ztile, *t3)
    t4 = _tc_d(p3, t3, dinv, b3, W4)
    p4 = _sc_spmm1(src, dst, ztile, t4)
    out = _tc_e(p4, t4, dinv, b4)
    return out[:N]


def kernel(x, edge_index, W1, b1, W2, b2, W3, b3, W4, b4):
    return _run(x, edge_index, W1, b1, W2, b2, W3, b3, W4, b4)
